# baseline (speedup 1.0000x reference)
import functools

import jax
import jax.numpy as jnp
from jax import lax
from jax.experimental import pallas as pl
from jax.experimental.pallas import tpu as pltpu
from jax.experimental.pallas import tpu_sc as plsc

_INPUT_DIM = 768
_NUM_EXPERTS = 64
_BLOCK_T = 4096
_CHUNK = 256
_GRP = 16

_NW = 32
_TOKENS = 32768
_TPW = _TOKENS // _NW
_GPW = _TPW // _GRP


def _dot(a, bm):
    return jax.lax.dot_general(
        a, bm,
        dimension_numbers=(((1,), (0,)), ((), ())),
        preferred_element_type=jnp.float32,
    )


def _router_block(x0_ref, x1_ref, x2_ref, wt_ref, b_ref, pk_ref):
    w = wt_ref[...]
    logits = (_dot(x0_ref[...], w[0:_CHUNK, :])
              + _dot(x1_ref[...], w[_CHUNK:2 * _CHUNK, :])
              + _dot(x2_ref[...], w[2 * _CHUNK:3 * _CHUNK, :])
              + b_ref[...])
    v1 = jnp.max(logits, axis=1, keepdims=True)
    m1 = logits == v1
    masked = jnp.where(m1, -jnp.inf, logits)
    v2 = jnp.max(masked, axis=1, keepdims=True)
    m2 = masked == v2
    t = jnp.exp(v2 - v1)
    w1 = 1.0 / (1.0 + t)
    w2 = t * w1
    colf = jax.lax.broadcasted_iota(
        jnp.int32, (_NUM_EXPERTS, 1), 0).astype(jnp.float32)
    i1f = _dot(m1.astype(jnp.float32), colf)
    i2f = _dot(m2.astype(jnp.float32), colf)
    g = _BLOCK_T // _GRP
    pk_ref[...] = jnp.concatenate(
        [w1.reshape(g, _GRP), w2.reshape(g, _GRP),
         i1f.reshape(g, _GRP), i2f.reshape(g, _GRP),
         jnp.zeros((g, 64), jnp.float32)], axis=1)


def _router(x, wt, b2):
    tokens = x.shape[0]
    grid = (tokens // _BLOCK_T,)
    xspec = lambda j: pl.BlockSpec((_BLOCK_T, _CHUNK), lambda i, j=j: (i, j))
    return pl.pallas_call(
        _router_block,
        grid=grid,
        in_specs=[
            xspec(0),
            xspec(1),
            xspec(2),
            pl.BlockSpec((_INPUT_DIM, _NUM_EXPERTS), lambda i: (0, 0)),
            pl.BlockSpec((1, _NUM_EXPERTS), lambda i: (0, 0)),
        ],
        out_specs=pl.BlockSpec((_BLOCK_T // _GRP, 128), lambda i: (i, 0)),
        out_shape=jax.ShapeDtypeStruct((tokens // _GRP, 128), jnp.float32),
        compiler_params=pltpu.CompilerParams(
            dimension_semantics=("parallel",),
        ),
    )(x, x, x, wt, b2)


@functools.partial(
    pl.kernel,
    out_type=jax.ShapeDtypeStruct((_TOKENS, _NUM_EXPERTS), jnp.float32),
    mesh=plsc.VectorSubcoreMesh(core_axis_name="c", subcore_axis_name="s"),
    scratch_types=[
        pltpu.VMEM((_GPW, 128), jnp.float32),
        pltpu.VMEM((_TPW // 2, _NUM_EXPERTS), jnp.float32),
    ],
    compiler_params=pltpu.CompilerParams(needs_layout_passes=False),
)
def _sc_expand(pk_hbm, out_hbm, pk_v, rows_v):
    wid = lax.axis_index("s") * 2 + lax.axis_index("c")
    pltpu.sync_copy(pk_hbm.at[pl.ds(wid * _GPW, _GPW), :], pk_v)
    zvec = jnp.zeros((_GRP,), jnp.float32)
    iota = lax.iota(jnp.int32, 16)
    half_rows = _TPW // 2
    half_groups = _GPW // 2

    for p in range(2):
        def zero_body(t, carry):
            rows_v[t, pl.ds(0, 16)] = zvec
            rows_v[t, pl.ds(16, 16)] = zvec
            rows_v[t, pl.ds(32, 16)] = zvec
            rows_v[t, pl.ds(48, 16)] = zvec
            return carry

        lax.fori_loop(0, half_rows, zero_body, 0)

        def scat_body(g, carry):
            gg = p * half_groups + g
            w1 = pk_v[gg, pl.ds(0, 16)]
            w2 = pk_v[gg, pl.ds(16, 16)]
            i1 = pk_v[gg, pl.ds(32, 16)].astype(jnp.int32)
            i2 = pk_v[gg, pl.ds(48, 16)].astype(jnp.int32)
            row = g * _GRP + iota
            plsc.store_scatter(rows_v, [row, i1], w1)
            plsc.store_scatter(rows_v, [row, i2], w2)
            return carry

        lax.fori_loop(0, half_groups, scat_body, 0)
        pltpu.sync_copy(
            rows_v,
            out_hbm.at[pl.ds(wid * _TPW + p * half_rows, half_rows), :])


@jax.jit
def kernel(x, W, b):
    wt = W.T
    b2 = b.reshape(1, _NUM_EXPERTS)
    pk = _router(x, wt, b2)
    return _sc_expand(pk)
